# Initial kernel scaffold; baseline (speedup 1.0000x reference)
import jax
import jax.numpy as jnp
from jax.experimental import pallas as pl


def kernel(x, source_mean_pe):
    raise NotImplementedError("write your pallas kernel here")


# speedup vs baseline: 2.5387x; 2.5387x over previous
import functools

import numpy as np
import jax
import jax.numpy as jnp
from jax import lax
from jax.experimental import pallas as pl
from jax.experimental.pallas import tpu as pltpu
from jax.experimental.pallas import tpu_sc as plsc


@functools.lru_cache(maxsize=None)
def _pe_table(S, d):
    pos = np.arange(S)[:, None].astype(np.float64)
    i = np.arange(d)[None, :]
    angle = pos / np.power(10000.0, (2 * (i // 2)).astype(np.float64) / d)
    pe = np.zeros((S, d), dtype=np.float64)
    pe[:, 0::2] = np.sin(angle[:, 0::2])
    pe[:, 1::2] = np.cos(angle[:, 1::2])
    return jnp.asarray(pe, dtype=jnp.float32)


def _dtw_tc_kernel(pe_ref, x_ref, mt_ref, dsel_ref, sel_ref, c_ref, dall_ref):
    S, d = pe_ref.shape
    SK = mt_ref.shape[0]
    K = SK // S
    INF = jnp.float32(jnp.inf)

    pes = x_ref[0] + pe_ref[...]
    p2 = jnp.sum(pes * pes, axis=1)

    RCH = 512
    for ci in range(SK // RCH):
        mchunk = mt_ref[pl.ds(ci * RCH, RCH), :]
        m2 = jnp.sum(mchunk * mchunk, axis=1, keepdims=True)
        cross = lax.dot_general(mchunk, pes, (((1,), (1,)), ((), ())),
                                preferred_element_type=jnp.float32)
        c_ref[pl.ds(ci * RCH, RCH)] = jnp.maximum(
            m2 + p2[None, :] - 2.0 * cross, 0.0)

    def row_step(g, prev):
        crow = c_ref[pl.ds(g * K, K)]
        d0 = jnp.where(g == 0, jnp.float32(0.0), INF)
        prev_sh = jnp.concatenate(
            [jnp.full((K, 1), d0, jnp.float32), prev[:, :-1]], axis=1)
        E = jnp.minimum(prev, prev_sh)
        P = crow
        sh = 1
        while sh < S:
            P = P + jnp.concatenate(
                [jnp.zeros((K, sh), jnp.float32), P[:, :-sh]], axis=1)
            sh *= 2
        G = E - (P - crow)
        H = G
        sh = 1
        while sh < S:
            H = jnp.minimum(H, jnp.concatenate(
                [jnp.full((K, sh), INF, jnp.float32), H[:, :-sh]], axis=1))
            sh *= 2
        drow = P + H
        dall_ref[pl.ds(g * K, K)] = drow
        return drow

    last = lax.fori_loop(0, S, row_step, jnp.full((K, S), INF, jnp.float32))

    dist = last[:, S - 1:S]
    mn = jnp.min(dist)
    iota_k = lax.broadcasted_iota(jnp.int32, (K, 1), 0)
    selb = jnp.min(jnp.where(dist == mn, iota_k, jnp.int32(1 << 30)))

    CH = 32
    for ci in range(S // CH):
        chunk = dall_ref[pl.ds(ci * CH * K, CH * K)].reshape(CH, K, S)
        pick = jnp.zeros((CH, S), jnp.float32)
        for k in range(K):
            w = jnp.where(selb == k, jnp.float32(1.0), jnp.float32(0.0))
            pick = pick + w * chunk[:, k, :]
        dsel_ref[0, pl.ds(ci * CH, CH), :] = lax.bitcast_convert_type(
            pick, jnp.int32)
    sel_ref[...] = jnp.full((1, 1, 16), selb, jnp.int32)


def _sc_backtrack_gather(dsel_hbm, sel_hbm, table_hbm, out_hbm,
                         db_v, sel_v, src_v, rows_v, sem):
    S, d = rows_v.shape
    B = dsel_hbm.shape[0]
    cid = lax.axis_index("c")
    sid = lax.axis_index("s")
    wid = sid * 2 + cid

    @pl.when(wid < B)
    def _():
        b = wid
        pltpu.sync_copy(dsel_hbm.at[b], db_v)
        pltpu.sync_copy(sel_hbm, sel_v)
        selv = sel_v[b]
        base = selv * S
        for c in range(src_v.shape[0]):
            for o in range(src_v.shape[1] // 16):
                src_v[c, pl.ds(o * 16, 16)] = base

        INF_I = jnp.int32(0x7F800000)
        lane0 = lax.iota(jnp.int32, 16) == 0

        def getv(ii, jj):
            inb = (ii >= 1) & (jj >= 1)
            i0 = jnp.maximum(ii - 1, 0)
            j0 = jnp.maximum(jj - 1, 0)
            v = plsc.load_gather(db_v, [i0, j0])
            bnd = jnp.where((ii == 0) & (jj == 0), jnp.int32(0), INF_I)
            return jnp.where(inb, v, bnd)

        def step(t, carry):
            iv, jv = carry
            active = (iv > 0) & (jv > 0)
            im1 = iv - 1
            jm1 = jv - 1
            o0 = getv(im1, jm1)
            o1 = getv(im1, jv)
            o2 = getv(iv, jm1)
            a0 = (o0 <= o1) & (o0 <= o2)
            a1 = jnp.logical_not(a0) & (o1 <= o2)
            is2 = jnp.logical_not(a0) & jnp.logical_not(a1)
            jidx = jnp.maximum(jm1, 0)
            cur = plsc.load_gather(src_v, [jidx >> 7, jidx & 127])
            new = jnp.maximum(cur, base + im1)
            plsc.store_scatter(src_v, [jidx >> 7, jidx & 127], new,
                               mask=lane0 & active)
            iv = jnp.where(active & jnp.logical_not(is2), im1, iv)
            jv = jnp.where(active & jnp.logical_not(a1), jm1, jv)
            return (iv, jv)

        sv = jnp.full((16,), S, jnp.int32)
        lax.fori_loop(0, 2 * S - 1, step, (sv, sv))

        for c in range(src_v.shape[0]):
            pltpu.async_copy(table_hbm.at[src_v.at[c]],
                             rows_v.at[pl.ds(c * 128, 128)], sem).wait()
        pltpu.sync_copy(rows_v, out_hbm.at[pl.ds(b * S, S)])


def kernel(x, source_mean_pe):
    B, S, d = x.shape
    K = source_mean_pe.shape[0]
    pe = _pe_table(S, d)

    meansT = jnp.transpose(source_mean_pe, (1, 0, 2)).reshape(S * K, d)
    dsel, sel = pl.pallas_call(
        _dtw_tc_kernel,
        grid=(B,),
        in_specs=[
            pl.BlockSpec((S, d), lambda b: (0, 0)),
            pl.BlockSpec((1, S, d), lambda b: (b, 0, 0)),
            pl.BlockSpec((S * K, d), lambda b: (0, 0)),
        ],
        out_specs=(pl.BlockSpec((1, S, S), lambda b: (b, 0, 0)),
                   pl.BlockSpec((1, 1, 16), lambda b: (b, 0, 0))),
        out_shape=(jax.ShapeDtypeStruct((B, S, S), jnp.int32),
                   jax.ShapeDtypeStruct((B, 1, 16), jnp.int32)),
        scratch_shapes=[
            pltpu.VMEM((S * K, S), jnp.float32),
            pltpu.VMEM((S * K, S), jnp.float32),
        ],
    )(pe, x, meansT)

    sel = sel.reshape(B, 16)
    table = source_mean_pe.reshape(K * S, d)
    mesh = plsc.VectorSubcoreMesh(core_axis_name="c", subcore_axis_name="s")
    flat = pl.kernel(
        _sc_backtrack_gather,
        out_type=jax.ShapeDtypeStruct((B * S, d), jnp.float32),
        mesh=mesh,
        compiler_params=pltpu.CompilerParams(needs_layout_passes=False),
        scratch_types=[
            pltpu.VMEM((S, S), jnp.int32),
            pltpu.VMEM((8, 16), jnp.int32),
            pltpu.VMEM((S // 128, 128), jnp.int32),
            pltpu.VMEM((S, d), jnp.float32),
            pltpu.SemaphoreType.DMA,
        ],
    )(dsel, sel, table)
    return flat.reshape(B, S, d)


# speedup vs baseline: 21.6814x; 8.5402x over previous
import functools

import numpy as np
import jax
import jax.numpy as jnp
from jax import lax
from jax.experimental import pallas as pl
from jax.experimental.pallas import tpu as pltpu
from jax.experimental.pallas import tpu_sc as plsc


@functools.lru_cache(maxsize=None)
def _pe_table(S, d):
    pos = np.arange(S)[:, None].astype(np.float64)
    i = np.arange(d)[None, :]
    angle = pos / np.power(10000.0, (2 * (i // 2)).astype(np.float64) / d)
    pe = np.zeros((S, d), dtype=np.float64)
    pe[:, 0::2] = np.sin(angle[:, 0::2])
    pe[:, 1::2] = np.cos(angle[:, 1::2])
    return jnp.asarray(pe, dtype=jnp.float32)


def _cost_prefix_kernel(pe_ref, x_ref, mt_ref, p_ref):
    S, d = pe_ref.shape
    SK = mt_ref.shape[0]
    K = SK // S

    pes = x_ref[0] + pe_ref[...]
    p2 = jnp.sum(pes * pes, axis=1)

    RCH = 256
    for ci in range(SK // RCH):
        mchunk = mt_ref[pl.ds(ci * RCH, RCH), :]
        m2 = jnp.sum(mchunk * mchunk, axis=1, keepdims=True)
        cross = lax.dot_general(mchunk, pes, (((1,), (1,)), ((), ())),
                                preferred_element_type=jnp.float32)
        cc = jnp.maximum(m2 + p2[None, :] - 2.0 * cross, 0.0)
        P = cc
        sh = 1
        while sh < S:
            P = P + jnp.concatenate(
                [jnp.zeros((RCH, sh), jnp.float32), P[:, :-sh]], axis=1)
            sh *= 2
        p_ref[pl.ds(ci * (RCH // K), RCH // K)] = P.reshape(RCH // K, K, S)


def _dtw_dp_kernel(p_ref, dall_ref, sel_ref):
    S, KB, _ = p_ref.shape
    B = sel_ref.shape[0]
    K = KB // B
    INF = jnp.float32(jnp.inf)

    def row_step(g, prev):
        P = p_ref[g]
        d0 = jnp.where(g == 0, jnp.float32(0.0), INF)
        prev_sh = jnp.concatenate(
            [jnp.full((KB, 1), d0, jnp.float32), prev[:, :-1]], axis=1)
        E = jnp.minimum(prev, prev_sh)
        Psh = jnp.concatenate(
            [jnp.zeros((KB, 1), jnp.float32), P[:, :-1]], axis=1)
        G = E - Psh
        H = G
        sh = 1
        while sh < S:
            H = jnp.minimum(H, jnp.concatenate(
                [jnp.full((KB, sh), INF, jnp.float32), H[:, :-sh]], axis=1))
            sh *= 2
        drow = P + H
        dall_ref[pl.ds(g * KB, KB)] = drow
        return drow

    last = lax.fori_loop(0, S, row_step, jnp.full((KB, S), INF, jnp.float32))

    dist = last[:, S - 1:S].reshape(B, K)
    mn = jnp.min(dist, axis=1, keepdims=True)
    io = lax.broadcasted_iota(jnp.int32, (B, K), 1)
    selv = jnp.min(jnp.where(dist == mn, io, jnp.int32(1 << 30)),
                   axis=1, keepdims=True)
    sel_ref[...] = jnp.broadcast_to(selv[:, :, None], (B, 1, 16))


def _sc_backtrack_gather(dall_hbm, sel_hbm, table_hbm, out_hbm,
                         db_v, sel_v, idxd_v, src_v, rows_v, sem):
    S, d = rows_v.shape
    B = sel_hbm.shape[0]
    KB = dall_hbm.shape[0] // S
    K = KB // B
    cid = lax.axis_index("c")
    sid = lax.axis_index("s")
    wid = sid * 2 + cid

    @pl.when(wid < B)
    def _():
        b = wid
        pltpu.sync_copy(sel_hbm, sel_v)
        selv = sel_v[b]
        lanes = lax.iota(jnp.int32, 16)

        base_row = b * K + selv
        for c in range(S // 16):
            idxd_v[c >> 3, pl.ds((c & 7) * 16, 16)] = (
                (c * 16 + lanes) * KB + base_row)
        for cc in range(S // 128):
            pltpu.async_copy(dall_hbm.at[idxd_v.at[cc]],
                             db_v.at[pl.ds(cc * 128, 128)], sem).wait()

        base = selv * S
        for c in range(src_v.shape[0]):
            for o in range(src_v.shape[1] // 16):
                src_v[c, pl.ds(o * 16, 16)] = base

        INF = jnp.full((16,), jnp.inf, jnp.float32)
        lane0 = lanes == 0

        def getv(ii, jj):
            inb = (ii >= 1) & (jj >= 1)
            i0 = jnp.maximum(ii - 1, 0)
            j0 = jnp.maximum(jj - 1, 0)
            v = plsc.load_gather(db_v, [i0, j0])
            bnd = jnp.where((ii == 0) & (jj == 0), jnp.float32(0.0), INF)
            return jnp.where(inb, v, bnd)

        def step(t, carry):
            iv, jv = carry
            active = (iv > 0) & (jv > 0)
            im1 = iv - 1
            jm1 = jv - 1
            o0 = getv(im1, jm1)
            o1 = getv(im1, jv)
            o2 = getv(iv, jm1)
            a0 = (o0 <= o1) & (o0 <= o2)
            a1 = jnp.logical_not(a0) & (o1 <= o2)
            is2 = jnp.logical_not(a0) & jnp.logical_not(a1)
            jidx = jnp.maximum(jm1, 0)
            cur = plsc.load_gather(src_v, [jidx >> 7, jidx & 127])
            new = jnp.maximum(cur, base + im1)
            plsc.store_scatter(src_v, [jidx >> 7, jidx & 127], new,
                               mask=lane0 & active)
            iv = jnp.where(active & jnp.logical_not(is2), im1, iv)
            jv = jnp.where(active & jnp.logical_not(a1), jm1, jv)
            return (iv, jv)

        sv = jnp.full((16,), S, jnp.int32)
        lax.fori_loop(0, 2 * S - 1, step, (sv, sv))

        for c in range(src_v.shape[0]):
            pltpu.async_copy(table_hbm.at[src_v.at[c]],
                             rows_v.at[pl.ds(c * 128, 128)], sem).wait()
        pltpu.sync_copy(rows_v, out_hbm.at[pl.ds(b * S, S)])


def kernel(x, source_mean_pe):
    B, S, d = x.shape
    K = source_mean_pe.shape[0]
    KB = K * B
    pe = _pe_table(S, d)

    meansT = jnp.transpose(source_mean_pe, (1, 0, 2)).reshape(S * K, d)
    p_all = pl.pallas_call(
        _cost_prefix_kernel,
        grid=(B,),
        in_specs=[
            pl.BlockSpec((S, d), lambda b: (0, 0)),
            pl.BlockSpec((1, S, d), lambda b: (b, 0, 0)),
            pl.BlockSpec((S * K, d), lambda b: (0, 0)),
        ],
        out_specs=pl.BlockSpec((S, K, S), lambda b: (0, b, 0)),
        out_shape=jax.ShapeDtypeStruct((S, KB, S), jnp.float32),
    )(pe, x, meansT)

    dall, sel = pl.pallas_call(
        _dtw_dp_kernel,
        out_shape=(jax.ShapeDtypeStruct((S * KB, S), jnp.float32),
                   jax.ShapeDtypeStruct((B, 1, 16), jnp.int32)),
    )(p_all)

    sel = sel.reshape(B, 16)
    table = source_mean_pe.reshape(K * S, d)
    mesh = plsc.VectorSubcoreMesh(core_axis_name="c", subcore_axis_name="s")
    flat = pl.kernel(
        _sc_backtrack_gather,
        out_type=jax.ShapeDtypeStruct((B * S, d), jnp.float32),
        mesh=mesh,
        compiler_params=pltpu.CompilerParams(needs_layout_passes=False),
        scratch_types=[
            pltpu.VMEM((S, S), jnp.float32),
            pltpu.VMEM((8, 16), jnp.int32),
            pltpu.VMEM((S // 128, 128), jnp.int32),
            pltpu.VMEM((S // 128, 128), jnp.int32),
            pltpu.VMEM((S, d), jnp.float32),
            pltpu.SemaphoreType.DMA,
        ],
    )(dall, sel, table)
    return flat.reshape(B, S, d)


# speedup vs baseline: 22.2923x; 1.0282x over previous
import functools

import numpy as np
import jax
import jax.numpy as jnp
from jax import lax
from jax.experimental import pallas as pl
from jax.experimental.pallas import tpu as pltpu
from jax.experimental.pallas import tpu_sc as plsc


@functools.lru_cache(maxsize=None)
def _pe_table(S, d):
    pos = np.arange(S)[:, None].astype(np.float64)
    i = np.arange(d)[None, :]
    angle = pos / np.power(10000.0, (2 * (i // 2)).astype(np.float64) / d)
    pe = np.zeros((S, d), dtype=np.float64)
    pe[:, 0::2] = np.sin(angle[:, 0::2])
    pe[:, 1::2] = np.cos(angle[:, 1::2])
    return jnp.asarray(pe, dtype=jnp.float32)


def _dtw_fused_kernel(pe_ref, x_ref, mt_ref, dall_ref, sel_ref, p_ref):
    S, d = pe_ref.shape
    SK = mt_ref.shape[0]
    K = SK // S
    B = p_ref.shape[0]
    KB = K * B
    pid = pl.program_id(0)
    INF = jnp.float32(jnp.inf)

    @pl.when(pid < B)
    def _stage1():
        pes = x_ref[0] + pe_ref[...]
        p2 = jnp.sum(pes * pes, axis=1)
        RCH = 256
        for ci in range(SK // RCH):
            mchunk = mt_ref[pl.ds(ci * RCH, RCH), :]
            m2 = jnp.sum(mchunk * mchunk, axis=1, keepdims=True)
            cross = lax.dot_general(mchunk, pes, (((1,), (1,)), ((), ())),
                                    preferred_element_type=jnp.float32)
            cc = jnp.maximum(m2 + p2[None, :] - 2.0 * cross, 0.0)
            P = cc
            sh = 1
            while sh < S:
                P = P + jnp.concatenate(
                    [jnp.zeros((RCH, sh), jnp.float32), P[:, :-sh]], axis=1)
                sh *= 2
            p_ref[pid, pl.ds(ci * RCH, RCH), :] = P

    @pl.when(pid == B)
    def _stage2():
        def row_step(g, prev):
            P = jnp.concatenate(
                [p_ref[bb, pl.ds(g * K, K), :] for bb in range(B)], axis=0)
            d0 = jnp.where(g == 0, jnp.float32(0.0), INF)
            prev_sh = jnp.concatenate(
                [jnp.full((KB, 1), d0, jnp.float32), prev[:, :-1]], axis=1)
            E = jnp.minimum(prev, prev_sh)
            Psh = jnp.concatenate(
                [jnp.zeros((KB, 1), jnp.float32), P[:, :-1]], axis=1)
            G = E - Psh
            H = G
            sh = 1
            while sh < S:
                H = jnp.minimum(H, jnp.concatenate(
                    [jnp.full((KB, sh), INF, jnp.float32), H[:, :-sh]],
                    axis=1))
                sh *= 2
            drow = P + H
            dall_ref[pl.ds(g * KB, KB)] = drow
            return drow

        last = lax.fori_loop(0, S, row_step,
                             jnp.full((KB, S), INF, jnp.float32))

        dist = last[:, S - 1:S].reshape(B, K)
        mn = jnp.min(dist, axis=1, keepdims=True)
        io = lax.broadcasted_iota(jnp.int32, (B, K), 1)
        selv = jnp.min(jnp.where(dist == mn, io, jnp.int32(1 << 30)),
                       axis=1, keepdims=True)
        sel_ref[...] = jnp.broadcast_to(selv[:, :, None], (B, 1, 16))


def _sc_backtrack_gather(dall_hbm, sel_hbm, table_hbm, out_hbm,
                         db_v, sel_v, idxd_v, src_v, rows_v, sem):
    S, d = rows_v.shape
    B = sel_hbm.shape[0]
    KB = dall_hbm.shape[0] // S
    K = KB // B
    cid = lax.axis_index("c")
    sid = lax.axis_index("s")
    wid = sid * 2 + cid

    @pl.when(wid < B)
    def _():
        b = wid
        pltpu.sync_copy(sel_hbm, sel_v)
        selv = sel_v[b]
        lanes = lax.iota(jnp.int32, 16)

        base_row = b * K + selv
        for c in range(S // 16):
            idxd_v[c >> 3, pl.ds((c & 7) * 16, 16)] = (
                (c * 16 + lanes) * KB + base_row)
        for cc in range(S // 128):
            pltpu.async_copy(dall_hbm.at[idxd_v.at[cc]],
                             db_v.at[pl.ds(cc * 128, 128)], sem).wait()

        base = selv * S
        for c in range(src_v.shape[0]):
            for o in range(src_v.shape[1] // 16):
                src_v[c, pl.ds(o * 16, 16)] = base

        INF = jnp.full((16,), jnp.inf, jnp.float32)
        lane0 = lanes == 0

        def getv(ii, jj):
            inb = (ii >= 1) & (jj >= 1)
            i0 = jnp.maximum(ii - 1, 0)
            j0 = jnp.maximum(jj - 1, 0)
            v = plsc.load_gather(db_v, [i0, j0])
            bnd = jnp.where((ii == 0) & (jj == 0), jnp.float32(0.0), INF)
            return jnp.where(inb, v, bnd)

        def step(t, carry):
            iv, jv = carry
            active = (iv > 0) & (jv > 0)
            im1 = iv - 1
            jm1 = jv - 1
            o0 = getv(im1, jm1)
            o1 = getv(im1, jv)
            o2 = getv(iv, jm1)
            a0 = (o0 <= o1) & (o0 <= o2)
            a1 = jnp.logical_not(a0) & (o1 <= o2)
            is2 = jnp.logical_not(a0) & jnp.logical_not(a1)
            jidx = jnp.maximum(jm1, 0)
            cur = plsc.load_gather(src_v, [jidx >> 7, jidx & 127])
            new = jnp.maximum(cur, base + im1)
            plsc.store_scatter(src_v, [jidx >> 7, jidx & 127], new,
                               mask=lane0 & active)
            iv = jnp.where(active & jnp.logical_not(is2), im1, iv)
            jv = jnp.where(active & jnp.logical_not(a1), jm1, jv)
            return (iv, jv)

        sv = jnp.full((16,), S, jnp.int32)
        lax.fori_loop(0, 2 * S - 1, step, (sv, sv))

        for c in range(src_v.shape[0]):
            pltpu.async_copy(table_hbm.at[src_v.at[c]],
                             rows_v.at[pl.ds(c * 128, 128)], sem).wait()
        pltpu.sync_copy(rows_v, out_hbm.at[pl.ds(b * S, S)])


def kernel(x, source_mean_pe):
    B, S, d = x.shape
    K = source_mean_pe.shape[0]
    KB = K * B
    pe = _pe_table(S, d)

    meansT = jnp.transpose(source_mean_pe, (1, 0, 2)).reshape(S * K, d)
    dall, sel = pl.pallas_call(
        _dtw_fused_kernel,
        grid=(B + 1,),
        in_specs=[
            pl.BlockSpec((S, d), lambda i: (0, 0)),
            pl.BlockSpec((1, S, d), lambda i: (i % B, 0, 0)),
            pl.BlockSpec((S * K, d), lambda i: (0, 0)),
        ],
        out_specs=(pl.BlockSpec((S * KB, S), lambda i: (0, 0)),
                   pl.BlockSpec((B, 1, 16), lambda i: (0, 0, 0))),
        out_shape=(jax.ShapeDtypeStruct((S * KB, S), jnp.float32),
                   jax.ShapeDtypeStruct((B, 1, 16), jnp.int32)),
        scratch_shapes=[pltpu.VMEM((B, S * K, S), jnp.float32)],
    )(pe, x, meansT)

    sel = sel.reshape(B, 16)
    table = source_mean_pe.reshape(K * S, d)
    mesh = plsc.VectorSubcoreMesh(core_axis_name="c", subcore_axis_name="s")
    flat = pl.kernel(
        _sc_backtrack_gather,
        out_type=jax.ShapeDtypeStruct((B * S, d), jnp.float32),
        mesh=mesh,
        compiler_params=pltpu.CompilerParams(needs_layout_passes=False),
        scratch_types=[
            pltpu.VMEM((S, S), jnp.float32),
            pltpu.VMEM((8, 16), jnp.int32),
            pltpu.VMEM((S // 128, 128), jnp.int32),
            pltpu.VMEM((S // 128, 128), jnp.int32),
            pltpu.VMEM((S, d), jnp.float32),
            pltpu.SemaphoreType.DMA,
        ],
    )(dall, sel, table)
    return flat.reshape(B, S, d)


# speedup vs baseline: 22.6651x; 1.0167x over previous
import functools

import numpy as np
import jax
import jax.numpy as jnp
from jax import lax
from jax.experimental import pallas as pl
from jax.experimental.pallas import tpu as pltpu
from jax.experimental.pallas import tpu_sc as plsc


@functools.lru_cache(maxsize=None)
def _pe_table(S, d):
    pos = np.arange(S)[:, None].astype(np.float64)
    i = np.arange(d)[None, :]
    angle = pos / np.power(10000.0, (2 * (i // 2)).astype(np.float64) / d)
    pe = np.zeros((S, d), dtype=np.float64)
    pe[:, 0::2] = np.sin(angle[:, 0::2])
    pe[:, 1::2] = np.cos(angle[:, 1::2])
    return jnp.asarray(pe, dtype=jnp.float32)


def _dtw_fused_kernel(pe_ref, x_ref, mt_ref, dall_ref, sel_ref, p_ref):
    S, d = pe_ref.shape
    SK = mt_ref.shape[0]
    K = SK // S
    B = p_ref.shape[0]
    KB = K * B
    pid = pl.program_id(0)
    INF = jnp.float32(jnp.inf)

    @pl.when(pid < B)
    def _stage1():
        pes = x_ref[0] + pe_ref[...]
        p2 = jnp.sum(pes * pes, axis=1)
        RCH = 256
        for ci in range(SK // RCH):
            mchunk = mt_ref[pl.ds(ci * RCH, RCH), :]
            m2 = jnp.sum(mchunk * mchunk, axis=1, keepdims=True)
            cross = lax.dot_general(mchunk, pes, (((1,), (1,)), ((), ())),
                                    preferred_element_type=jnp.float32)
            cc = jnp.maximum(m2 + p2[None, :] - 2.0 * cross, 0.0)
            P = cc
            sh = 1
            while sh < S:
                P = P + jnp.concatenate(
                    [jnp.zeros((RCH, sh), jnp.float32), P[:, :-sh]], axis=1)
                sh *= 2
            p_ref[pid, pl.ds(ci * RCH, RCH), :] = P

    @pl.when(pid == B)
    def _stage2():
        def one_row(g, prev):
            P = jnp.concatenate(
                [p_ref[bb, pl.ds(g * K, K), :] for bb in range(B)], axis=0)
            d0 = jnp.where(g == 0, jnp.float32(0.0), INF)
            prev_sh = jnp.concatenate(
                [jnp.full((KB, 1), d0, jnp.float32), prev[:, :-1]], axis=1)
            E = jnp.minimum(prev, prev_sh)
            Psh = jnp.concatenate(
                [jnp.zeros((KB, 1), jnp.float32), P[:, :-1]], axis=1)
            G = E - Psh
            H = G
            sh = 1
            while sh < S:
                H = jnp.minimum(H, jnp.concatenate(
                    [jnp.full((KB, sh), INF, jnp.float32), H[:, :-sh]],
                    axis=1))
                sh *= 2
            drow = P + H
            dall_ref[pl.ds(g * KB, KB)] = drow
            return drow

        def row_pair(gg, prev):
            return one_row(gg * 2 + 1, one_row(gg * 2, prev))

        last = lax.fori_loop(0, S // 2, row_pair,
                             jnp.full((KB, S), INF, jnp.float32))

        dist = last[:, S - 1:S].reshape(B, K)
        mn = jnp.min(dist, axis=1, keepdims=True)
        io = lax.broadcasted_iota(jnp.int32, (B, K), 1)
        selv = jnp.min(jnp.where(dist == mn, io, jnp.int32(1 << 30)),
                       axis=1, keepdims=True)
        sel_ref[...] = jnp.broadcast_to(selv[:, :, None], (B, 1, 16))


def _sc_backtrack_gather(dall_hbm, sel_hbm, table_hbm, out_hbm,
                         db_v, sel_v, idxd_v, src_v, rows_v, sem):
    S, d = rows_v.shape
    B = sel_hbm.shape[0]
    KB = dall_hbm.shape[0] // S
    K = KB // B
    cid = lax.axis_index("c")
    sid = lax.axis_index("s")
    wid = sid * 2 + cid

    @pl.when(wid < B)
    def _():
        b = wid
        pltpu.sync_copy(sel_hbm, sel_v)
        selv = sel_v[b]
        lanes = lax.iota(jnp.int32, 16)

        base_row = b * K + selv
        for c in range(S // 16):
            idxd_v[c >> 3, pl.ds((c & 7) * 16, 16)] = (
                (c * 16 + lanes) * KB + base_row)
        for cc in range(S // 128):
            pltpu.async_copy(dall_hbm.at[idxd_v.at[cc]],
                             db_v.at[pl.ds(cc * 128, 128)], sem).wait()

        base = selv * S
        for c in range(src_v.shape[0]):
            for o in range(src_v.shape[1] // 16):
                src_v[c, pl.ds(o * 16, 16)] = base

        INF = jnp.full((16,), jnp.inf, jnp.float32)
        lane0 = lanes == 0

        def getv(ii, jj):
            inb = (ii >= 1) & (jj >= 1)
            i0 = jnp.maximum(ii - 1, 0)
            j0 = jnp.maximum(jj - 1, 0)
            v = plsc.load_gather(db_v, [i0, j0])
            bnd = jnp.where((ii == 0) & (jj == 0), jnp.float32(0.0), INF)
            return jnp.where(inb, v, bnd)

        def step(t, carry):
            iv, jv = carry
            active = (iv > 0) & (jv > 0)
            im1 = iv - 1
            jm1 = jv - 1
            o0 = getv(im1, jm1)
            o1 = getv(im1, jv)
            o2 = getv(iv, jm1)
            a0 = (o0 <= o1) & (o0 <= o2)
            a1 = jnp.logical_not(a0) & (o1 <= o2)
            is2 = jnp.logical_not(a0) & jnp.logical_not(a1)
            jidx = jnp.maximum(jm1, 0)
            cur = plsc.load_gather(src_v, [jidx >> 7, jidx & 127])
            new = jnp.maximum(cur, base + im1)
            plsc.store_scatter(src_v, [jidx >> 7, jidx & 127], new,
                               mask=lane0 & active)
            iv = jnp.where(active & jnp.logical_not(is2), im1, iv)
            jv = jnp.where(active & jnp.logical_not(a1), jm1, jv)
            return (iv, jv)

        sv = jnp.full((16,), S, jnp.int32)
        lax.fori_loop(0, 2 * S - 1, step, (sv, sv))

        for c in range(src_v.shape[0]):
            pltpu.async_copy(table_hbm.at[src_v.at[c]],
                             rows_v.at[pl.ds(c * 128, 128)], sem).wait()
        pltpu.sync_copy(rows_v, out_hbm.at[pl.ds(b * S, S)])


def kernel(x, source_mean_pe):
    B, S, d = x.shape
    K = source_mean_pe.shape[0]
    KB = K * B
    pe = _pe_table(S, d)

    meansT = jnp.transpose(source_mean_pe, (1, 0, 2)).reshape(S * K, d)
    dall, sel = pl.pallas_call(
        _dtw_fused_kernel,
        grid=(B + 1,),
        in_specs=[
            pl.BlockSpec((S, d), lambda i: (0, 0)),
            pl.BlockSpec((1, S, d), lambda i: (i % B, 0, 0)),
            pl.BlockSpec((S * K, d), lambda i: (0, 0)),
        ],
        out_specs=(pl.BlockSpec((S * KB, S), lambda i: (0, 0)),
                   pl.BlockSpec((B, 1, 16), lambda i: (0, 0, 0))),
        out_shape=(jax.ShapeDtypeStruct((S * KB, S), jnp.float32),
                   jax.ShapeDtypeStruct((B, 1, 16), jnp.int32)),
        scratch_shapes=[pltpu.VMEM((B, S * K, S), jnp.float32)],
    )(pe, x, meansT)

    sel = sel.reshape(B, 16)
    table = source_mean_pe.reshape(K * S, d)
    mesh = plsc.VectorSubcoreMesh(core_axis_name="c", subcore_axis_name="s")
    flat = pl.kernel(
        _sc_backtrack_gather,
        out_type=jax.ShapeDtypeStruct((B * S, d), jnp.float32),
        mesh=mesh,
        compiler_params=pltpu.CompilerParams(needs_layout_passes=False),
        scratch_types=[
            pltpu.VMEM((S, S), jnp.float32),
            pltpu.VMEM((8, 16), jnp.int32),
            pltpu.VMEM((S // 128, 128), jnp.int32),
            pltpu.VMEM((S // 128, 128), jnp.int32),
            pltpu.VMEM((S, d), jnp.float32),
            pltpu.SemaphoreType.DMA,
        ],
    )(dall, sel, table)
    return flat.reshape(B, S, d)
